# baseline (speedup 1.0000x reference)
import functools

import jax
import jax.numpy as jnp
from jax import lax
from jax.experimental import pallas as pl
from jax.experimental.pallas import tpu as pltpu
from jax.experimental.pallas import tpu_sc as plsc

N = 10000
N_ACC = 10240
NC = 2
NSUB = 16
CHUNK = 128
CPT = 160
E_PAD = NSUB * CPT * CHUNK
ROWS_PER_TILE = N_ACC // NSUB
DEG_W = 16
DEG_CPT = 80

_MESH = plsc.VectorSubcoreMesh(core_axis_name="c", subcore_axis_name="s")


def _make_sc_scatter(DH):
    @functools.partial(
        pl.kernel,
        out_type=jax.ShapeDtypeStruct((NC, N_ACC, DH), jnp.float32),
        mesh=_MESH,
        scratch_types=[
            pltpu.VMEM((CPT, CHUNK), jnp.int32),
            pltpu.VMEM((CPT, CHUNK), jnp.int32),
            pltpu.VMEM((4, CHUNK, DH), jnp.float32),
            pltpu.VMEM_SHARED((N_ACC, DH), jnp.float32),
            [pltpu.SemaphoreType.DMA] * 4,
            [pltpu.SemaphoreType.DMA] * 4,
        ],
        compiler_params=pltpu.CompilerParams(use_tc_tiling_on_sc=False),
    )
    def scat(table, src3, dst3, zrows, out, src_v, dst_v, rows_v, acc,
             gsems, ssems):
        c = lax.axis_index("c")
        s = lax.axis_index("s")
        row0 = s * ROWS_PER_TILE
        tbl = table.at[c]
        pltpu.sync_copy(zrows, acc.at[pl.ds(row0, ROWS_PER_TILE)])
        pltpu.sync_copy(src3.at[s], src_v)
        pltpu.sync_copy(dst3.at[s], dst_v)
        plsc.subcore_barrier()

        def gcp(j, slot):
            return pltpu.make_async_copy(
                tbl.at[src_v.at[j]], rows_v.at[slot], gsems[slot])

        def scp(j, slot):
            return pltpu.make_async_copy(
                rows_v.at[slot], acc.at[dst_v.at[j]], ssems[slot])

        def sstart(j, slot):
            pltpu.async_copy(rows_v.at[slot], acc.at[dst_v.at[j]],
                             ssems[slot], add=True)

        gcp(0, 0).start()
        gcp(1, 1).start()
        gcp(2, 2).start()
        gcp(0, 0).wait()
        sstart(0, 0)
        gcp(3, 3).start()
        gcp(1, 1).wait()
        sstart(1, 1)

        def body(g, carry):
            j0 = 2 + g * 4
            for i in range(4):
                j = j0 + i
                slot = (2 + i) % 4
                nslot = (slot + 2) % 4
                scp(j - 2, nslot).wait()
                gcp(j + 2, nslot).start()
                gcp(j, slot).wait()
                sstart(j, slot)
            return carry

        lax.fori_loop(0, (CPT - 4) // 4, body, 0)
        for j, slot in ((CPT - 2, (CPT - 2) % 4), (CPT - 1, (CPT - 1) % 4)):
            scp(j - 2, (slot + 2) % 4).wait()
            gcp(j, slot).wait()
            sstart(j, slot)
        scp(CPT - 2, (CPT - 2) % 4).wait()
        scp(CPT - 1, (CPT - 1) % 4).wait()
        plsc.subcore_barrier()
        pltpu.sync_copy(acc.at[pl.ds(row0, ROWS_PER_TILE)],
                        out.at[c, pl.ds(row0, ROWS_PER_TILE)])

    return scat


@functools.partial(
    pl.kernel,
    out_type=jax.ShapeDtypeStruct((N_ACC, 128), jnp.float32),
    mesh=_MESH,
    scratch_types=[
        pltpu.VMEM((DEG_CPT, CHUNK), jnp.int32),
        pltpu.VMEM((CHUNK, DEG_W), jnp.float32),
        pltpu.VMEM_SHARED((N_ACC, DEG_W), jnp.float32),
    ],
    compiler_params=pltpu.CompilerParams(use_tc_tiling_on_sc=False),
)
def _sc_degree(ones_rows, dst3, zrows, out, dst_v, ones_v, acc):
    c = lax.axis_index("c")
    s = lax.axis_index("s")
    wid = s * NC + c
    row0 = s * ROWS_PER_TILE
    pltpu.sync_copy(zrows, acc.at[pl.ds(row0, ROWS_PER_TILE)])
    pltpu.sync_copy(dst3.at[wid], dst_v)
    pltpu.sync_copy(ones_rows, ones_v)
    plsc.subcore_barrier()

    def body(j, carry):
        pltpu.sync_copy(ones_v, acc.at[dst_v.at[j]], add=True)
        return carry

    lax.fori_loop(0, DEG_CPT, body, 0)
    plsc.subcore_barrier()
    pltpu.sync_copy(acc.at[pl.ds(row0, ROWS_PER_TILE)],
                    out.at[pl.ds(row0, ROWS_PER_TILE),
                           pl.ds(c * DEG_W, DEG_W)])


_BLK = 400


def _rows128():
    return pl.BlockSpec((_BLK, 128), lambda i: (i, 0))


def _whole(shape):
    return pl.BlockSpec(shape, lambda i: (0,) * len(shape))


def _dinv_of(dp_ref):
    deg = dp_ref[:, :1] + dp_ref[:, 16:17] + 1.0
    return lax.rsqrt(deg)


def _tc_first(x, w, dp):
    def body(x_ref, w_ref, dp_ref, o_ref):
        dinv = _dinv_of(dp_ref)
        o_ref[...] = jnp.dot(x_ref[...], w_ref[...],
                             preferred_element_type=jnp.float32) * dinv

    return pl.pallas_call(
        body,
        grid=(N // _BLK,),
        in_specs=[_rows128(), _whole(w.shape), _rows128()],
        out_specs=_rows128(),
        out_shape=jax.ShapeDtypeStruct((N, 128), jnp.float32),
    )(x, w, dp)


def _sblk(DH):
    return pl.BlockSpec((NC, _BLK, DH), lambda i: (0, i, 0))


def _tc_mid(s1, g1, dp, b, w):
    def body(s_ref, g_ref, dp_ref, b_ref, w_ref, o_ref):
        dinv = _dinv_of(dp_ref)
        xa = jnp.maximum(dinv * (s_ref[0] + g_ref[:, :64])
                         + b_ref[:, :64], 0.0)
        xb = jnp.maximum(dinv * (s_ref[1] + g_ref[:, 64:])
                         + b_ref[:, 64:], 0.0)
        g2 = (jnp.dot(xa, w_ref[:64], preferred_element_type=jnp.float32)
              + jnp.dot(xb, w_ref[64:], preferred_element_type=jnp.float32))
        o_ref[:, :64] = g2 * dinv

    return pl.pallas_call(
        body,
        grid=(N // _BLK,),
        in_specs=[_sblk(64), _rows128(), _rows128(), _whole(b.shape),
                  _whole(w.shape)],
        out_specs=_rows128(),
        out_shape=jax.ShapeDtypeStruct((N, 128), jnp.float32),
    )(s1, g1, dp, b, w)


def _tc_mid2(s2, g2, dp, b, w):
    def body(s_ref, g_ref, dp_ref, b_ref, w_ref, h_ref, o_ref):
        dinv = _dinv_of(dp_ref)
        ha = jnp.maximum(dinv * (s_ref[0] + g_ref[:, :32])
                         + b_ref[:, :32], 0.0)
        hb = jnp.maximum(dinv * (s_ref[1] + g_ref[:, 32:64])
                         + b_ref[:, 32:], 0.0)
        h_ref[:, :32] = ha
        h_ref[:, 32:] = hb
        g3 = (jnp.dot(ha, w_ref[:32], preferred_element_type=jnp.float32)
              + jnp.dot(hb, w_ref[32:], preferred_element_type=jnp.float32))
        o_ref[:, :64] = g3 * dinv

    return pl.pallas_call(
        body,
        grid=(N // _BLK,),
        in_specs=[_sblk(32), _rows128(), _rows128(), _whole(b.shape),
                  _whole(w.shape)],
        out_specs=[pl.BlockSpec((_BLK, 64), lambda i: (i, 0)), _rows128()],
        out_shape=[jax.ShapeDtypeStruct((N, 64), jnp.float32),
                   jax.ShapeDtypeStruct((N, 128), jnp.float32)],
    )(s2, g2, dp, b, w)


def _tc_last(s3, g3, dp, b):
    def body(s_ref, g_ref, dp_ref, b_ref, o_ref):
        dinv = _dinv_of(dp_ref)
        o_ref[:, :32] = dinv * (s_ref[0] + g_ref[:, :32]) + b_ref[:, :32]
        o_ref[:, 32:] = dinv * (s_ref[1] + g_ref[:, 32:64]) + b_ref[:, 32:]

    return pl.pallas_call(
        body,
        grid=(N // _BLK,),
        in_specs=[_sblk(32), _rows128(), _rows128(), _whole(b.shape)],
        out_specs=pl.BlockSpec((_BLK, 64), lambda i: (i, 0)),
        out_shape=jax.ShapeDtypeStruct((N, 64), jnp.float32),
    )(s3, g3, dp, b)


_scatter_h = _make_sc_scatter(64)
_scatter_q = _make_sc_scatter(32)


def kernel(x, edge_index, W1, b1, W2, b2, W3, b3):
    src = edge_index[0]
    dst = edge_index[1]
    pad_e = E_PAD - src.shape[0]
    src_p = jnp.concatenate([src, jnp.zeros((pad_e,), src.dtype)])
    dst_p = jnp.concatenate([dst, jnp.full((pad_e,), N, dst.dtype)])
    src3 = src_p.reshape(NSUB, CPT, CHUNK)
    dst3 = dst_p.reshape(NSUB, CPT, CHUNK)
    dstd = dst_p.reshape(NC * NSUB, DEG_CPT, CHUNK)

    ones_rows = jnp.ones((CHUNK, DEG_W), jnp.float32)
    z16 = jnp.zeros((ROWS_PER_TILE, DEG_W), jnp.float32)
    z64 = jnp.zeros((ROWS_PER_TILE, 64), jnp.float32)
    z32 = jnp.zeros((ROWS_PER_TILE, 32), jnp.float32)

    dp = _sc_degree(ones_rows, dstd, z16)

    g1 = _tc_first(x, W1, dp)
    gt1 = jnp.stack([g1[:, :64], g1[:, 64:]])
    s1 = _scatter_h(gt1, src3, dst3, z64)
    g2 = _tc_mid(s1, g1, dp, b1.reshape(1, -1), W2)
    gt2 = jnp.stack([g2[:, :32], g2[:, 32:64]])
    s2 = _scatter_q(gt2, src3, dst3, z32)
    h, g3 = _tc_mid2(s2, g2, dp, b2.reshape(1, -1), W3)
    gt3 = jnp.stack([g3[:, :32], g3[:, 32:64]])
    s3 = _scatter_q(gt3, src3, dst3, z32)
    out = _tc_last(s3, g3, dp, b3.reshape(1, -1))
    return (out, h)


# speedup vs baseline: 1.1675x; 1.1675x over previous
import functools

import jax
import jax.numpy as jnp
from jax import lax
from jax.experimental import pallas as pl
from jax.experimental.pallas import tpu as pltpu
from jax.experimental.pallas import tpu_sc as plsc

N = 10000
N_PAD = 10240
NC = 2
NSUB = 16
CHUNK = 128
CPT = 160
E_PAD = NSUB * CPT * CHUNK
ROWS_PER_TILE = N_PAD // NSUB
DEG_W = 16
DEG_CPT = 80

_MESH = plsc.VectorSubcoreMesh(core_axis_name="c", subcore_axis_name="s")


def _make_sc_scatter(DH):
    @functools.partial(
        pl.kernel,
        out_type=jax.ShapeDtypeStruct((NC, N_PAD, DH), jnp.float32),
        mesh=_MESH,
        scratch_types=[
            pltpu.VMEM((CPT, CHUNK), jnp.int32),
            pltpu.VMEM((CPT, CHUNK), jnp.int32),
            pltpu.VMEM((4, CHUNK, DH), jnp.float32),
            pltpu.VMEM_SHARED((N_PAD, DH), jnp.float32),
            [pltpu.SemaphoreType.DMA] * 4,
            [pltpu.SemaphoreType.DMA] * 4,
        ],
        compiler_params=pltpu.CompilerParams(use_tc_tiling_on_sc=False),
    )
    def scat(table, src3, dst3, zrows, out, src_v, dst_v, rows_v, acc,
             gsems, ssems):
        c = lax.axis_index("c")
        s = lax.axis_index("s")
        row0 = s * ROWS_PER_TILE
        tbl = table.at[c]
        pltpu.sync_copy(zrows, acc.at[pl.ds(row0, ROWS_PER_TILE)])
        pltpu.sync_copy(src3.at[s], src_v)
        pltpu.sync_copy(dst3.at[s], dst_v)
        plsc.subcore_barrier()

        def gcp(j, slot):
            return pltpu.make_async_copy(
                tbl.at[src_v.at[j]], rows_v.at[slot], gsems[slot])

        def scp(j, slot):
            return pltpu.make_async_copy(
                rows_v.at[slot], acc.at[dst_v.at[j]], ssems[slot])

        def sstart(j, slot):
            pltpu.async_copy(rows_v.at[slot], acc.at[dst_v.at[j]],
                             ssems[slot], add=True)

        gcp(0, 0).start()
        gcp(1, 1).start()
        gcp(2, 2).start()
        gcp(0, 0).wait()
        sstart(0, 0)
        gcp(3, 3).start()
        gcp(1, 1).wait()
        sstart(1, 1)

        def body(g, carry):
            j0 = 2 + g * 4
            for i in range(4):
                j = j0 + i
                slot = (2 + i) % 4
                nslot = (slot + 2) % 4
                scp(j - 2, nslot).wait()
                gcp(j + 2, nslot).start()
                gcp(j, slot).wait()
                sstart(j, slot)
            return carry

        lax.fori_loop(0, (CPT - 4) // 4, body, 0)
        for j, slot in ((CPT - 2, (CPT - 2) % 4), (CPT - 1, (CPT - 1) % 4)):
            scp(j - 2, (slot + 2) % 4).wait()
            gcp(j, slot).wait()
            sstart(j, slot)
        scp(CPT - 2, (CPT - 2) % 4).wait()
        scp(CPT - 1, (CPT - 1) % 4).wait()
        plsc.subcore_barrier()
        pltpu.sync_copy(acc.at[pl.ds(row0, ROWS_PER_TILE)],
                        out.at[c, pl.ds(row0, ROWS_PER_TILE)])

    return scat


@functools.partial(
    pl.kernel,
    out_type=jax.ShapeDtypeStruct((NC, N_PAD, DEG_W), jnp.float32),
    mesh=_MESH,
    scratch_types=[
        pltpu.VMEM((DEG_CPT, CHUNK), jnp.int32),
        pltpu.VMEM((CHUNK, DEG_W), jnp.float32),
        pltpu.VMEM_SHARED((N_PAD, DEG_W), jnp.float32),
    ],
    compiler_params=pltpu.CompilerParams(use_tc_tiling_on_sc=False),
)
def _sc_degree(ones_rows, dst3, zrows, out, dst_v, ones_v, acc):
    c = lax.axis_index("c")
    s = lax.axis_index("s")
    wid = s * NC + c
    row0 = s * ROWS_PER_TILE
    pltpu.sync_copy(zrows, acc.at[pl.ds(row0, ROWS_PER_TILE)])
    pltpu.sync_copy(dst3.at[wid], dst_v)
    pltpu.sync_copy(ones_rows, ones_v)
    plsc.subcore_barrier()

    def body(j, carry):
        pltpu.sync_copy(ones_v, acc.at[dst_v.at[j]], add=True)
        return carry

    lax.fori_loop(0, DEG_CPT, body, 0)
    plsc.subcore_barrier()
    pltpu.sync_copy(acc.at[pl.ds(row0, ROWS_PER_TILE)],
                    out.at[c, pl.ds(row0, ROWS_PER_TILE)])


_BLK = 1024


def _rows(D):
    return pl.BlockSpec((_BLK, D), lambda i: (i, 0))


def _crows(DH):
    return pl.BlockSpec((NC, _BLK, DH), lambda i: (0, i, 0))


def _whole(shape):
    return pl.BlockSpec(shape, lambda i: (0,) * len(shape))


def _dinv_of(d0_ref, d1_ref):
    deg = d0_ref[:, :1] + d1_ref[:, :1] + 1.0
    return lax.rsqrt(deg)


def _tc_first(x, w, d0, d1):
    def body(x_ref, w_ref, d0_ref, d1_ref, o_ref):
        dinv = _dinv_of(d0_ref, d1_ref)
        xv = x_ref[...]
        o_ref[0] = jnp.dot(xv, w_ref[:, :64],
                           preferred_element_type=jnp.float32) * dinv
        o_ref[1] = jnp.dot(xv, w_ref[:, 64:],
                           preferred_element_type=jnp.float32) * dinv

    return pl.pallas_call(
        body,
        grid=(N_PAD // _BLK,),
        in_specs=[_rows(128), _whole(w.shape), _rows(DEG_W), _rows(DEG_W)],
        out_specs=_crows(64),
        out_shape=jax.ShapeDtypeStruct((NC, N_PAD, 64), jnp.float32),
    )(x, w, d0, d1)


def _tc_mid(s1, g1, d0, d1, b, w):
    def body(s_ref, g_ref, d0_ref, d1_ref, b_ref, w_ref, o_ref):
        dinv = _dinv_of(d0_ref, d1_ref)
        xa = jnp.maximum(dinv * (s_ref[0] + g_ref[0]) + b_ref[:, :64], 0.0)
        xb = jnp.maximum(dinv * (s_ref[1] + g_ref[1]) + b_ref[:, 64:], 0.0)
        ga = jnp.dot(xa, w_ref[:64], preferred_element_type=jnp.float32)
        gb = jnp.dot(xb, w_ref[64:], preferred_element_type=jnp.float32)
        gsum = ga + gb
        o_ref[0] = gsum[:, :32] * dinv
        o_ref[1] = gsum[:, 32:] * dinv

    return pl.pallas_call(
        body,
        grid=(N_PAD // _BLK,),
        in_specs=[_crows(64), _crows(64), _rows(DEG_W), _rows(DEG_W),
                  _whole(b.shape), _whole(w.shape)],
        out_specs=_crows(32),
        out_shape=jax.ShapeDtypeStruct((NC, N_PAD, 32), jnp.float32),
    )(s1, g1, d0, d1, b, w)


def _tc_mid2(s2, g2, d0, d1, b, w):
    def body(s_ref, g_ref, d0_ref, d1_ref, b_ref, w_ref, h_ref, o_ref):
        dinv = _dinv_of(d0_ref, d1_ref)
        ha = jnp.maximum(dinv * (s_ref[0] + g_ref[0]) + b_ref[:, :32], 0.0)
        hb = jnp.maximum(dinv * (s_ref[1] + g_ref[1]) + b_ref[:, 32:], 0.0)
        h_ref[:, :32] = ha
        h_ref[:, 32:] = hb
        gsum = (jnp.dot(ha, w_ref[:32], preferred_element_type=jnp.float32)
                + jnp.dot(hb, w_ref[32:], preferred_element_type=jnp.float32))
        o_ref[0] = gsum[:, :32] * dinv
        o_ref[1] = gsum[:, 32:] * dinv

    return pl.pallas_call(
        body,
        grid=(N_PAD // _BLK,),
        in_specs=[_crows(32), _crows(32), _rows(DEG_W), _rows(DEG_W),
                  _whole(b.shape), _whole(w.shape)],
        out_specs=[_rows(64), _crows(32)],
        out_shape=[jax.ShapeDtypeStruct((N_PAD, 64), jnp.float32),
                   jax.ShapeDtypeStruct((NC, N_PAD, 32), jnp.float32)],
    )(s2, g2, d0, d1, b, w)


def _tc_last(s3, g3, d0, d1, b):
    def body(s_ref, g_ref, d0_ref, d1_ref, b_ref, o_ref):
        dinv = _dinv_of(d0_ref, d1_ref)
        o_ref[:, :32] = dinv * (s_ref[0] + g_ref[0]) + b_ref[:, :32]
        o_ref[:, 32:] = dinv * (s_ref[1] + g_ref[1]) + b_ref[:, 32:]

    return pl.pallas_call(
        body,
        grid=(N_PAD // _BLK,),
        in_specs=[_crows(32), _crows(32), _rows(DEG_W), _rows(DEG_W),
                  _whole(b.shape)],
        out_specs=_rows(64),
        out_shape=jax.ShapeDtypeStruct((N_PAD, 64), jnp.float32),
    )(s3, g3, d0, d1, b)


_scatter64 = _make_sc_scatter(64)
_scatter32 = _make_sc_scatter(32)


def kernel(x, edge_index, W1, b1, W2, b2, W3, b3):
    src = edge_index[0]
    dst = edge_index[1]
    pad_e = E_PAD - src.shape[0]
    src_p = jnp.concatenate([src, jnp.zeros((pad_e,), src.dtype)])
    dst_p = jnp.concatenate([dst, jnp.full((pad_e,), N, dst.dtype)])
    src3 = src_p.reshape(NSUB, CPT, CHUNK)
    dst3 = dst_p.reshape(NSUB, CPT, CHUNK)
    srcd = src_p.reshape(NC * NSUB, DEG_CPT, CHUNK)
    dstd = dst_p.reshape(NC * NSUB, DEG_CPT, CHUNK)
    del srcd
    x_pad = jnp.zeros((N_PAD, x.shape[1]), jnp.float32).at[:N].set(x)

    ones_rows = jnp.ones((CHUNK, DEG_W), jnp.float32)
    z16 = jnp.zeros((ROWS_PER_TILE, DEG_W), jnp.float32)
    z64 = jnp.zeros((ROWS_PER_TILE, 64), jnp.float32)
    z32 = jnp.zeros((ROWS_PER_TILE, 32), jnp.float32)

    degp = _sc_degree(ones_rows, dstd, z16)
    d0, d1 = degp[0], degp[1]

    g1 = _tc_first(x_pad, W1, d0, d1)
    s1 = _scatter64(g1, src3, dst3, z64)
    g2 = _tc_mid(s1, g1, d0, d1, b1.reshape(1, -1), W2)
    s2 = _scatter32(g2, src3, dst3, z32)
    h_full, g3 = _tc_mid2(s2, g2, d0, d1, b2.reshape(1, -1), W3)
    s3 = _scatter32(g3, src3, dst3, z32)
    out_full = _tc_last(s3, g3, d0, d1, b3.reshape(1, -1))
    return (out_full[:N], h_full[:N])
